# baseline (speedup 1.0000x reference)
import numpy as np
import jax
import jax.numpy as jnp
from jax.sharding import Mesh, PartitionSpec as P
from jax.experimental import pallas as pl
from jax.experimental.pallas import tpu as pltpu

try:
    from jax import shard_map as _shard_map_fn
except ImportError:
    from jax.experimental.shard_map import shard_map as _shard_map_fn

_DIM = 4096
_INTER = 11008
_BM = 512
_BN = 256


def _fused_mlp_kernel(x_ref, w1_ref, w3_ref, w2_ref, y_ref):
    j = pl.program_id(1)

    @pl.when(j == 0)
    def _init():
        y_ref[...] = jnp.zeros_like(y_ref)

    x = x_ref[...]
    h1 = jax.lax.dot_general(x, w1_ref[...], (((1,), (1,)), ((), ())),
                             preferred_element_type=jnp.float32)
    h3 = jax.lax.dot_general(x, w3_ref[...], (((1,), (1,)), ((), ())),
                             preferred_element_type=jnp.float32)
    g = (h1 * jax.lax.logistic(h1) * h3).astype(jnp.bfloat16)
    y_ref[...] += jax.lax.dot_general(g, w2_ref[...], (((1,), (1,)), ((), ())),
                                      preferred_element_type=jnp.float32)


def _fused_mlp(xb, w1b, w3b, w2b):
    tokens = xb.shape[0]
    grid = (tokens // _BM, _INTER // _BN)
    return pl.pallas_call(
        _fused_mlp_kernel,
        grid=grid,
        in_specs=[
            pl.BlockSpec((_BM, _DIM), lambda i, j: (i, 0)),
            pl.BlockSpec((_BN, _DIM), lambda i, j: (j, 0)),
            pl.BlockSpec((_BN, _DIM), lambda i, j: (j, 0)),
            pl.BlockSpec((_DIM, _BN), lambda i, j: (0, j)),
        ],
        out_specs=pl.BlockSpec((_BM, _DIM), lambda i, j: (i, 0)),
        out_shape=jax.ShapeDtypeStruct((tokens, _DIM), jnp.float32),
        compiler_params=pltpu.CompilerParams(
            dimension_semantics=("parallel", "arbitrary"),
        ),
    )(xb, w1b, w3b, w2b)


def kernel(x, w1, w2, w3):
    xb = x.astype(jnp.bfloat16)
    w1b = w1.astype(jnp.bfloat16)
    w3b = w3.astype(jnp.bfloat16)
    w2b = w2.astype(jnp.bfloat16)
    devs = jax.devices()
    if len(devs) >= 2 and x.shape[0] % (2 * _BM) == 0:
        mesh = Mesh(np.array(devs[:2]), ("d",))
        fn = _shard_map_fn(
            _fused_mlp, mesh=mesh,
            in_specs=(P("d", None), P(None, None), P(None, None), P(None, None)),
            out_specs=P("d", None), check_vma=False)
        return fn(xb, w1b, w3b, w2b)
    return _fused_mlp(xb, w1b, w3b, w2b)


# speedup vs baseline: 1.0107x; 1.0107x over previous
import numpy as np
import jax
import jax.numpy as jnp
from jax.sharding import Mesh, PartitionSpec as P
from jax.experimental import pallas as pl
from jax.experimental.pallas import tpu as pltpu

try:
    from jax import shard_map as _shard_map_fn
except ImportError:
    from jax.experimental.shard_map import shard_map as _shard_map_fn

_DIM = 4096
_INTER = 11008
_BM = 1024
_BN = 256


def _fused_mlp_kernel(x_ref, w1_ref, w3_ref, w2_ref, y_ref):
    j = pl.program_id(1)

    @pl.when(j == 0)
    def _init():
        y_ref[...] = jnp.zeros_like(y_ref)

    x = x_ref[...]
    h1 = jax.lax.dot_general(x, w1_ref[...], (((1,), (1,)), ((), ())),
                             preferred_element_type=jnp.float32)
    h3 = jax.lax.dot_general(x, w3_ref[...], (((1,), (1,)), ((), ())),
                             preferred_element_type=jnp.float32)
    g = (h1 * jax.lax.logistic(h1) * h3).astype(jnp.bfloat16)
    y_ref[...] += jax.lax.dot_general(g, w2_ref[...], (((1,), (1,)), ((), ())),
                                      preferred_element_type=jnp.float32)


def _fused_mlp(xb, w1b, w3b, w2b):
    tokens = xb.shape[0]
    grid = (tokens // _BM, _INTER // _BN)
    return pl.pallas_call(
        _fused_mlp_kernel,
        grid=grid,
        in_specs=[
            pl.BlockSpec((_BM, _DIM), lambda i, j: (i, 0)),
            pl.BlockSpec((_BN, _DIM), lambda i, j: (j, 0)),
            pl.BlockSpec((_BN, _DIM), lambda i, j: (j, 0)),
            pl.BlockSpec((_DIM, _BN), lambda i, j: (0, j)),
        ],
        out_specs=pl.BlockSpec((_BM, _DIM), lambda i, j: (i, 0)),
        out_shape=jax.ShapeDtypeStruct((tokens, _DIM), jnp.float32),
        compiler_params=pltpu.CompilerParams(
            dimension_semantics=("parallel", "arbitrary"),
            vmem_limit_bytes=62 * 1024 * 1024,
        ),
    )(xb, w1b, w3b, w2b)


def kernel(x, w1, w2, w3):
    xb = x.astype(jnp.bfloat16)
    w1b = w1.astype(jnp.bfloat16)
    w3b = w3.astype(jnp.bfloat16)
    w2b = w2.astype(jnp.bfloat16)
    devs = jax.devices()
    if len(devs) >= 2 and x.shape[0] % (2 * _BM) == 0:
        mesh = Mesh(np.array(devs[:2]), ("d",))
        fn = _shard_map_fn(
            _fused_mlp, mesh=mesh,
            in_specs=(P("d", None), P(None, None), P(None, None), P(None, None)),
            out_specs=P("d", None), check_vma=False)
        return fn(xb, w1b, w3b, w2b)
    return _fused_mlp(xb, w1b, w3b, w2b)


# speedup vs baseline: 1.1096x; 1.0979x over previous
import numpy as np
import jax
import jax.numpy as jnp
from jax.sharding import Mesh, PartitionSpec as P
from jax.experimental import pallas as pl
from jax.experimental.pallas import tpu as pltpu

try:
    from jax import shard_map as _shard_map_fn
except ImportError:
    from jax.experimental.shard_map import shard_map as _shard_map_fn

_DIM = 4096
_INTER = 11008
_BM = 1024
_BN = 256


def _fused_mlp_kernel(x_ref, w1_ref, w3_ref, w2_ref, y_ref):
    j = pl.program_id(1)

    @pl.when(j == 0)
    def _init():
        y_ref[...] = jnp.zeros_like(y_ref)

    x = x_ref[...]
    h1 = jax.lax.dot_general(x, w1_ref[...], (((1,), (1,)), ((), ())),
                             preferred_element_type=jnp.float32)
    h3 = jax.lax.dot_general(x, w3_ref[...], (((1,), (1,)), ((), ())),
                             preferred_element_type=jnp.float32)
    g = (h1 * jax.lax.logistic(h1) * h3).astype(jnp.bfloat16)
    y_ref[...] += jax.lax.dot_general(g, w2_ref[...], (((1,), (1,)), ((), ())),
                                      preferred_element_type=jnp.float32)


def _fused_mlp(xb, w1b, w3b, w2b):
    tokens = xb.shape[0]
    grid = (tokens // _BM, _INTER // _BN)
    return pl.pallas_call(
        _fused_mlp_kernel,
        grid=grid,
        in_specs=[
            pl.BlockSpec((_BM, _DIM), lambda i, j: (i, 0)),
            pl.BlockSpec((_BN, _DIM), lambda i, j: (j, 0)),
            pl.BlockSpec((_BN, _DIM), lambda i, j: (j, 0)),
            pl.BlockSpec((_DIM, _BN), lambda i, j: (0, j)),
        ],
        out_specs=pl.BlockSpec((_BM, _DIM), lambda i, j: (i, 0)),
        out_shape=jax.ShapeDtypeStruct((tokens, _DIM), jnp.float32),
        compiler_params=pltpu.CompilerParams(
            dimension_semantics=("parallel", "arbitrary"),
            vmem_limit_bytes=62 * 1024 * 1024,
        ),
    )(xb, w1b, w3b, w2b)


def _sharded_mlp(x_loc, w1_half, w3_half, w2_half):
    xb = x_loc.astype(jnp.bfloat16)
    w1b = jax.lax.all_gather(w1_half.astype(jnp.bfloat16), "d", axis=0,
                             tiled=True)
    w3b = jax.lax.all_gather(w3_half.astype(jnp.bfloat16), "d", axis=0,
                             tiled=True)
    w2b = jax.lax.all_gather(w2_half.astype(jnp.bfloat16), "d", axis=1,
                             tiled=True)
    return _fused_mlp(xb, w1b, w3b, w2b)


def kernel(x, w1, w2, w3):
    devs = jax.devices()
    if len(devs) >= 2 and x.shape[0] % (2 * _BM) == 0:
        mesh = Mesh(np.array(devs[:2]), ("d",))
        fn = _shard_map_fn(
            _sharded_mlp, mesh=mesh,
            in_specs=(P("d", None), P("d", None), P("d", None), P(None, "d")),
            out_specs=P("d", None), check_vma=False)
        return fn(x, w1, w3, w2)
    return _fused_mlp(x.astype(jnp.bfloat16), w1.astype(jnp.bfloat16),
                      w3.astype(jnp.bfloat16), w2.astype(jnp.bfloat16))


# speedup vs baseline: 1.1470x; 1.0337x over previous
import numpy as np
import jax
import jax.numpy as jnp
from jax.sharding import Mesh, PartitionSpec as P
from jax.experimental import pallas as pl
from jax.experimental.pallas import tpu as pltpu

try:
    from jax import shard_map as _shard_map_fn
except ImportError:
    from jax.experimental.shard_map import shard_map as _shard_map_fn

_DIM = 4096
_INTER = 11008
_BM = 1024
_BN = 256


def _fused_mlp_kernel(x_ref, w1_ref, w3_ref, w2_ref, y_ref):
    j = pl.program_id(1)

    @pl.when(j == 0)
    def _init():
        y_ref[...] = jnp.zeros_like(y_ref)

    x = x_ref[...]
    h1 = jax.lax.dot_general(x, w1_ref[...], (((1,), (1,)), ((), ())),
                             preferred_element_type=jnp.float32)
    h3 = jax.lax.dot_general(x, w3_ref[...], (((1,), (1,)), ((), ())),
                             preferred_element_type=jnp.float32)
    g = (h1 * jax.lax.logistic(h1) * h3).astype(jnp.bfloat16)
    y_ref[...] += jax.lax.dot_general(g, w2_ref[...], (((1,), (1,)), ((), ())),
                                      preferred_element_type=jnp.float32)


def _fused_mlp(xb, w1b, w3b, w2b):
    tokens = xb.shape[0]
    grid = (tokens // _BM, _INTER // _BN)
    return pl.pallas_call(
        _fused_mlp_kernel,
        grid=grid,
        in_specs=[
            pl.BlockSpec((_BM, _DIM), lambda i, j: (i, 0)),
            pl.BlockSpec((_BN, _DIM), lambda i, j: (j, 0)),
            pl.BlockSpec((_BN, _DIM), lambda i, j: (j, 0)),
            pl.BlockSpec((_DIM, _BN), lambda i, j: (0, j)),
        ],
        out_specs=pl.BlockSpec((_BM, _DIM), lambda i, j: (i, 0)),
        out_shape=jax.ShapeDtypeStruct((tokens, _DIM), jnp.float32),
        compiler_params=pltpu.CompilerParams(
            dimension_semantics=("parallel", "arbitrary"),
            vmem_limit_bytes=62 * 1024 * 1024,
        ),
    )(xb, w1b, w3b, w2b)


def _sharded_mlp(xb_loc, w1_half, w3_half, w2_half):
    w1b = jax.lax.all_gather(w1_half, "d", axis=0, tiled=True)
    w3b = jax.lax.all_gather(w3_half, "d", axis=0, tiled=True)
    w2b = jax.lax.all_gather(w2_half, "d", axis=1, tiled=True)
    return _fused_mlp(xb_loc, w1b, w3b, w2b)


def kernel(x, w1, w2, w3):
    xb = x.astype(jnp.bfloat16)
    w1b = w1.astype(jnp.bfloat16)
    w3b = w3.astype(jnp.bfloat16)
    w2b = w2.astype(jnp.bfloat16)
    devs = jax.devices()
    if len(devs) >= 2 and x.shape[0] % (2 * _BM) == 0:
        mesh = Mesh(np.array(devs[:2]), ("d",))
        fn = _shard_map_fn(
            _sharded_mlp, mesh=mesh,
            in_specs=(P("d", None), P("d", None), P("d", None), P(None, "d")),
            out_specs=P("d", None), check_vma=False)
        return fn(xb, w1b, w3b, w2b)
    return _fused_mlp(xb, w1b, w3b, w2b)
